# baseline (speedup 1.0000x reference)
import functools

import jax
import jax.numpy as jnp
from jax import lax
from jax.experimental import pallas as pl
from jax.experimental.pallas import tpu as pltpu
from jax.experimental.pallas import tpu_sc as plsc

N = 10000
E = 320000
D = 128
EPS = 1e-5
NC, NS = 2, 16
NW = NC * NS
CHUNK = 128
NCH = 80
EPT = NCH * CHUNK
EPAD = NW * EPT
NP = 10240
RPT = NP // NS
PH = 40

BLK = 2000
GRID = N // BLK


def _mm_body(h_ref, w_ref, b_ref, o_ref):
    o_ref[...] = (
        jnp.dot(h_ref[...], w_ref[...], preferred_element_type=jnp.float32)
        + b_ref[...]
    )


def _matmul(h, W, b):
    return pl.pallas_call(
        _mm_body,
        grid=(GRID,),
        in_specs=[
            pl.BlockSpec((BLK, D), lambda i: (i, 0)),
            pl.BlockSpec((D, D), lambda i: (0, 0)),
            pl.BlockSpec((1, D), lambda i: (0, 0)),
        ],
        out_specs=pl.BlockSpec((BLK, D), lambda i: (i, 0)),
        out_shape=jax.ShapeDtypeStruct((N, D), jnp.float32),
    )(h, W, b.reshape(1, D))


def _stats_body(a_ref, o_ref, acc_ref):
    i = pl.program_id(0)

    @pl.when(i == 0)
    def _():
        acc_ref[...] = jnp.zeros_like(acc_ref)

    x = a_ref[0] + a_ref[1]
    acc_ref[0:1] += jnp.sum(x, axis=0, keepdims=True)
    acc_ref[1:2] += jnp.sum(x * x, axis=0, keepdims=True)

    @pl.when(i == GRID - 1)
    def _():
        o_ref[...] = acc_ref[...]


def _stats(agg2):
    return pl.pallas_call(
        _stats_body,
        grid=(GRID,),
        in_specs=[pl.BlockSpec((2, BLK, D), lambda i: (0, i, 0))],
        out_specs=pl.BlockSpec((8, D), lambda i: (0, 0)),
        out_shape=jax.ShapeDtypeStruct((8, D), jnp.float32),
        scratch_shapes=[pltpu.VMEM((8, D), jnp.float32)],
    )(agg2)


def _bn_relu(st_ref, a_ref, g_ref, be_ref):
    mean = st_ref[0:1] / N
    var = st_ref[1:2] / N - mean * mean
    scale = g_ref[...] * lax.rsqrt(var + EPS)
    x = a_ref[0] + a_ref[1]
    return jnp.maximum((x - mean) * scale + be_ref[...], 0.0)


def _bn_body(st_ref, a_ref, g_ref, be_ref, o_ref):
    o_ref[...] = _bn_relu(st_ref, a_ref, g_ref, be_ref)


def _bn(st, agg2, g, be):
    return pl.pallas_call(
        _bn_body,
        grid=(GRID,),
        in_specs=[
            pl.BlockSpec((8, D), lambda i: (0, 0)),
            pl.BlockSpec((2, BLK, D), lambda i: (0, i, 0)),
            pl.BlockSpec((1, D), lambda i: (0, 0)),
            pl.BlockSpec((1, D), lambda i: (0, 0)),
        ],
        out_specs=pl.BlockSpec((BLK, D), lambda i: (i, 0)),
        out_shape=jax.ShapeDtypeStruct((N, D), jnp.float32),
    )(st, agg2, g.reshape(1, D), be.reshape(1, D))


def _bn_mm_body(st_ref, a_ref, g_ref, be_ref, w_ref, b_ref, o_ref):
    h = _bn_relu(st_ref, a_ref, g_ref, be_ref)
    o_ref[...] = (
        jnp.dot(h, w_ref[...], preferred_element_type=jnp.float32) + b_ref[...]
    )


def _bn_mm(st, agg2, g, be, Wn, bn):
    return pl.pallas_call(
        _bn_mm_body,
        grid=(GRID,),
        in_specs=[
            pl.BlockSpec((8, D), lambda i: (0, 0)),
            pl.BlockSpec((2, BLK, D), lambda i: (0, i, 0)),
            pl.BlockSpec((1, D), lambda i: (0, 0)),
            pl.BlockSpec((1, D), lambda i: (0, 0)),
            pl.BlockSpec((D, D), lambda i: (0, 0)),
            pl.BlockSpec((1, D), lambda i: (0, 0)),
        ],
        out_specs=pl.BlockSpec((BLK, D), lambda i: (i, 0)),
        out_shape=jax.ShapeDtypeStruct((N, D), jnp.float32),
    )(st, agg2, g.reshape(1, D), be.reshape(1, D), Wn, bn.reshape(1, D))


_sc_mesh = plsc.VectorSubcoreMesh(core_axis_name="c", subcore_axis_name="s")

TH = 5120
C = 32
MB = 128
NBLK = EPT // MB
TPAIR = EPT // (2 * C)
RWA = 632


@functools.partial(
    pl.kernel,
    out_type=jax.ShapeDtypeStruct((NC, N, D), jnp.float32),
    mesh=_sc_mesh,
    scratch_types=[
        pltpu.VMEM((2, MB), jnp.int32),
        pltpu.VMEM((2, MB // C, C), jnp.int32),
        pltpu.VMEM((2, MB), jnp.float32),
        pltpu.VMEM((C, D), jnp.float32),
        pltpu.VMEM((C, D), jnp.float32),
        pltpu.VMEM_SHARED((TH, D), jnp.float32),
        pltpu.VMEM_SHARED((N, D), jnp.float32),
        pltpu.SemaphoreType.DMA,
        pltpu.SemaphoreType.DMA,
        pltpu.SemaphoreType.DMA,
        pltpu.SemaphoreType.DMA,
        pltpu.SemaphoreType.DMA,
    ],
)
def _sc_edge(hlin_p, src1, dst1, w1, zeros, out,
             srcm, dstm, wm, rows0, rows1, table, acc, g0, g1, s0, s1, msem):
    c = lax.axis_index("c")
    s = lax.axis_index("s")
    wid = c * NS + s
    ebase = wid * EPT

    @pl.when(s < NS - 1)
    def _():
        pltpu.sync_copy(zeros, acc.at[pl.ds(s * RWA, RWA)])

    @pl.when(s == NS - 1)
    def _():
        pltpu.sync_copy(zeros.at[pl.ds(0, 520)], acc.at[pl.ds(9480, 520)])

    def fire_meta(blk, buf):
        base = ebase + blk * MB
        pltpu.async_copy(src1.at[pl.ds(base, MB)], srcm.at[buf], msem)
        pltpu.async_copy(w1.at[pl.ds(base, MB)], wm.at[buf], msem)
        for kk in range(MB // C):
            pltpu.async_copy(dst1.at[pl.ds(base + kk * C, C)], dstm.at[buf, kk], msem)

    def wait_meta(buf):
        pltpu.make_async_copy(src1.at[pl.ds(0, MB)], srcm.at[buf], msem).wait()
        pltpu.make_async_copy(w1.at[pl.ds(0, MB)], wm.at[buf], msem).wait()
        for kk in range(MB // C):
            pltpu.make_async_copy(dst1.at[pl.ds(0, C)], dstm.at[buf, kk], msem).wait()

    def fire_gather(mbuf, k, buf, sem):
        pltpu.async_copy(table.at[srcm.at[mbuf, pl.ds(k * C, C)]], buf, sem)

    def wait_gather(buf, sem):
        pltpu.make_async_copy(table.at[srcm.at[0, pl.ds(0, C)]], buf, sem).wait()

    def fire_scatter(mbuf, k, buf, sem):
        pltpu.async_copy(buf, acc.at[dstm.at[mbuf, k]], sem, add=True)

    def wait_scatter(buf, sem):
        pltpu.make_async_copy(buf, acc.at[dstm.at[0, 0]], sem).wait()

    for st in range(2):
        lo = st * TH

        def prep(mbuf, k):
            for h in range(C // 16):
                sl = pl.ds(k * C + h * 16, 16)
                sv = srcm[mbuf, sl]
                il = sv - lo
                valid = (il >= 0) & (il < TH)
                srcm[mbuf, sl] = jnp.where(valid, il, sv & 4095)
                wv = wm[mbuf, sl]
                wm[mbuf, sl] = jnp.where(valid, wv, 0.0)

        def scale(mbuf, k, buf):
            for h in range(C // 16):
                w16 = wm[mbuf, pl.ds(k * C + h * 16, 16)]
                for r in range(16):
                    wsc = w16[r]
                    i = h * 16 + r
                    for kk in range(D // 16):
                        sl2 = pl.ds(kk * 16, 16)
                        buf[i, sl2] = buf[i, sl2] * wsc

        plsc.subcore_barrier()
        pltpu.sync_copy(hlin_p.at[pl.ds(lo + s * 320, 320)], table.at[pl.ds(s * 320, 320)])
        plsc.subcore_barrier()

        fire_meta(0, 0)
        wait_meta(0)
        fire_meta(1, 1)
        prep(0, 0)
        fire_gather(0, 0, rows0, g0)

        @pl.loop(0, TPAIR)
        def _pair(t):
            j0 = 2 * t
            b = t // 2
            mb = b % 2
            k0 = j0 - b * 4
            k1 = k0 + 1
            @pl.when(t > 0)
            def _():
                wait_scatter(rows1, s1)

            @pl.when(jnp.logical_and(t % 2 == 0, t > 0))
            def _():
                @pl.when(b + 1 < NBLK)
                def _():
                    fire_meta(b + 1, 1 - mb)

            prep(mb, k1)
            fire_gather(mb, k1, rows1, g1)
            wait_gather(rows0, g0)
            scale(mb, k0, rows0)
            fire_scatter(mb, k0, rows0, s0)

            @pl.when(t < TPAIR - 1)
            def _():
                wait_scatter(rows0, s0)

                @pl.when(t % 2 == 1)
                def _():
                    wait_meta(1 - mb)

                nb = (t + 1) // 2
                nmb = nb % 2
                nk = (j0 + 2) - nb * 4
                prep(nmb, nk)
                fire_gather(nmb, nk, rows0, g0)

            wait_gather(rows1, g1)
            scale(mb, k1, rows1)
            fire_scatter(mb, k1, rows1, s1)

        wait_scatter(rows0, s0)
        wait_scatter(rows1, s1)

    plsc.subcore_barrier()

    @pl.when(s < NS - 1)
    def _():
        pltpu.sync_copy(acc.at[pl.ds(s * RWA, RWA)], out.at[c, pl.ds(s * RWA, RWA)])

    @pl.when(s == NS - 1)
    def _():
        pltpu.sync_copy(acc.at[pl.ds(9480, 520)], out.at[c, pl.ds(9480, 520)])


def kernel(node_features, edge_index, edges_weight,
           W0, b0, g0, be0, W1, b1, g1, be1, W2, b2, g2, be2):
    pad = EPAD - E
    src1 = jnp.pad(edge_index[0], (0, pad))
    dst1 = jnp.pad(edge_index[1], (0, pad))
    w1 = jnp.pad(edges_weight, (0, pad))
    zeros = jnp.zeros((RWA, D), jnp.float32)

    params = [(W0, b0, g0, be0), (W1, b1, g1, be1), (W2, b2, g2, be2)]
    hlin = _matmul(node_features, W0, b0)
    out = None
    for li in range(3):
        g, be = params[li][2], params[li][3]
        hlin_p = jnp.pad(hlin, ((0, 2 * TH - N), (0, 0)))
        agg2 = _sc_edge(hlin_p, src1, dst1, w1, zeros)
        st = _stats(agg2)
        if li < 2:
            Wn, bn = params[li + 1][0], params[li + 1][1]
            hlin = _bn_mm(st, agg2, g, be, Wn, bn)
        else:
            out = _bn(st, agg2, g, be)
    return out
